# baseline (speedup 1.0000x reference)
import functools

import jax
import jax.numpy as jnp
from jax import lax
from jax.experimental import pallas as pl
from jax.experimental.pallas import tpu as pltpu
from jax.experimental.pallas import tpu_sc as plsc

VOCAB = 100000
EMBED = 64
B = 1024
L = 50
LP = 56

NC = 2
NS = 16
NW = NC * NS
BPW = B // NW

VB = 2048


IPW = BPW * L
NCH = 4
RPC = BPW // NCH
IPC = RPC * L


def _pool_body(cw_hbm, embp_hbm, out_hbm, idx_v, rows0, rows1, acc_v,
               sem0, sem1):
    wid = lax.axis_index("s") * NC + lax.axis_index("c")
    base = wid * BPW
    pltpu.sync_copy(cw_hbm.at[pl.ds(wid * IPW, IPW)], idx_v)

    bufs = (rows0, rows1)
    sems = (sem0, sem1)

    def fire(c):
        return pltpu.async_copy(
            embp_hbm.at[idx_v.at[pl.ds(c * IPC, IPC)]],
            bufs[c % 2], sems[c % 2])

    copies = [None] * NCH
    copies[0] = fire(0)
    copies[1] = fire(1)
    for c in range(NCH):
        copies[c].wait()
        buf = bufs[c % 2]

        def row_body(r, carry, buf=buf, c=c):
            for j in range(EMBED // 16):
                acc = buf[L * r, pl.ds(16 * j, 16)]
                for i in range(1, L):
                    acc = acc + buf[L * r + i, pl.ds(16 * j, 16)]
                acc_v[c * RPC + r, pl.ds(16 * j, 16)] = acc
            return carry

        lax.fori_loop(0, RPC, row_body, 0)
        if c + 2 < NCH:
            copies[c + 2] = fire(c + 2)

    pltpu.sync_copy(acc_v, out_hbm.at[pl.ds(base, BPW)])


@jax.jit
def _pool(cw_flat, embp):
    return pl.kernel(
        _pool_body,
        out_type=jax.ShapeDtypeStruct((B, EMBED), jnp.float32),
        mesh=plsc.VectorSubcoreMesh(core_axis_name="c", subcore_axis_name="s"),
        scratch_types=[
            pltpu.VMEM((IPW,), jnp.int32),
            pltpu.VMEM((IPC, 2 * EMBED), jnp.float32),
            pltpu.VMEM((IPC, 2 * EMBED), jnp.float32),
            pltpu.VMEM((BPW, EMBED), jnp.float32),
            pltpu.SemaphoreType.DMA,
            pltpu.SemaphoreType.DMA,
        ],
        compiler_params=pltpu.CompilerParams(use_tc_tiling_on_sc=False),
    )(cw_flat, embp)


VBK = 1024


def _padt_body(embt_ref, out_ref):
    eye = jnp.eye(EMBED, dtype=jnp.float32)
    t = lax.dot_general(
        embt_ref[...], eye,
        (((0,), (0,)), ((), ())),
        preferred_element_type=jnp.float32,
    )
    out_ref[...] = jnp.pad(t, ((0, 0), (0, EMBED)))


@jax.jit
def _padt(embt):
    grid = (pl.cdiv(VOCAB, VBK),)
    return pl.pallas_call(
        _padt_body,
        grid=grid,
        in_specs=[pl.BlockSpec((EMBED, VBK), lambda i: (0, i))],
        out_specs=pl.BlockSpec((VBK, 2 * EMBED), lambda i: (i, 0)),
        out_shape=jax.ShapeDtypeStruct((VOCAB, 2 * EMBED), jnp.float32),
    )(embt)


def _proj_body(x_ref, wt_ref, b_ref, outt_ref, xn_ref):
    @pl.when(pl.program_id(0) == 0)
    def _():
        x = x_ref[...]
        mu = jnp.mean(x, axis=0, keepdims=True)
        xc = x - mu
        var = jnp.mean(xc * xc, axis=0, keepdims=True)
        xn_ref[...] = xc * lax.rsqrt(var + 1e-10)

    acc = lax.dot_general(
        wt_ref[...], xn_ref[...],
        (((0,), (1,)), ((), ())),
        preferred_element_type=jnp.float32,
    )
    bias = lax.dot_general(
        b_ref[...], jnp.ones((1, B), jnp.float32),
        (((0,), (0,)), ((), ())),
        preferred_element_type=jnp.float32,
    )
    outt_ref[...] = acc + bias


@jax.jit
def _proj(pooled, wt, b2d):
    grid = (pl.cdiv(VOCAB, VB),)
    return pl.pallas_call(
        _proj_body,
        grid=grid,
        in_specs=[
            pl.BlockSpec((B, EMBED), lambda i: (0, 0)),
            pl.BlockSpec((EMBED, VB), lambda i: (0, i)),
            pl.BlockSpec((1, VB), lambda i: (0, i)),
        ],
        out_specs=pl.BlockSpec((VB, B), lambda i: (i, 0)),
        out_shape=jax.ShapeDtypeStruct((VOCAB, B), jnp.float32),
        scratch_shapes=[pltpu.VMEM((B, EMBED), jnp.float32)],
    )(pooled, wt, b2d)


def kernel(context_words, emb, W, b):
    cw_flat = context_words.astype(jnp.int32).reshape(B * L)
    embp = _padt(emb.T)
    pooled = _pool(cw_flat, embp)
    outt = _proj(pooled, W.T, b.reshape(1, VOCAB))
    return outt.T


# speedup vs baseline: 1.2943x; 1.2943x over previous
import functools

import jax
import jax.numpy as jnp
from jax import lax
from jax.experimental import pallas as pl
from jax.experimental.pallas import tpu as pltpu
from jax.experimental.pallas import tpu_sc as plsc

VOCAB = 100000
EMBED = 64
B = 1024
L = 50

NC = 2
NS = 16
NW = NC * NS
DPW = EMBED // NW

CB = 256
VB = 2048


def _poolt_body(cwt_hbm, embt_hbm, out_hbm, row_v, cw_v, acc_v, sem):
    wid = lax.axis_index("s") * NC + lax.axis_index("c")
    for dl in range(DPW):
        d = DPW * wid + dl
        pltpu.sync_copy(embt_hbm.at[d], row_v)
        for bc in range(B // CB):
            pltpu.sync_copy(cwt_hbm.at[:, pl.ds(bc * CB, CB)], cw_v)

            def gbody(g, carry, dl=dl, bc=bc):
                acc = jnp.zeros((16,), jnp.float32)
                for i in range(L):
                    idx = cw_v[i, pl.ds(16 * g, 16)]
                    acc = acc + plsc.load_gather(row_v, [idx])
                acc_v[dl, pl.ds(bc * CB + 16 * g, 16)] = acc
                return carry

            lax.fori_loop(0, CB // 16, gbody, 0)
    pltpu.sync_copy(acc_v, out_hbm.at[pl.ds(DPW * wid, DPW)])


@jax.jit
def _poolt(cwt, embt):
    return pl.kernel(
        _poolt_body,
        out_type=jax.ShapeDtypeStruct((EMBED, B), jnp.float32),
        mesh=plsc.VectorSubcoreMesh(core_axis_name="c", subcore_axis_name="s"),
        scratch_types=[
            pltpu.VMEM((VOCAB,), jnp.float32),
            pltpu.VMEM((L, CB), jnp.int32),
            pltpu.VMEM((DPW, B), jnp.float32),
            pltpu.SemaphoreType.DMA,
        ],
        compiler_params=pltpu.CompilerParams(
            use_tc_tiling_on_sc=True, needs_layout_passes=False),
    )(cwt, embt)


def _proj_body(xt_ref, wt_ref, b_ref, outt_ref, xn_ref):
    @pl.when(pl.program_id(0) == 0)
    def _():
        xt = xt_ref[...]
        mu = jnp.mean(xt, axis=1, keepdims=True)
        xc = xt - mu
        var = jnp.mean(xc * xc, axis=1, keepdims=True)
        xn_ref[...] = xc * lax.rsqrt(var + 1e-10)

    acc = lax.dot_general(
        wt_ref[...], xn_ref[...],
        (((0,), (0,)), ((), ())),
        preferred_element_type=jnp.float32,
    )
    bias = lax.dot_general(
        b_ref[...], jnp.ones((1, B), jnp.float32),
        (((0,), (0,)), ((), ())),
        preferred_element_type=jnp.float32,
    )
    outt_ref[...] = acc + bias


@jax.jit
def _proj(xt, wt, b2d):
    grid = (pl.cdiv(VOCAB, VB),)
    return pl.pallas_call(
        _proj_body,
        grid=grid,
        in_specs=[
            pl.BlockSpec((EMBED, B), lambda i: (0, 0)),
            pl.BlockSpec((EMBED, VB), lambda i: (0, i)),
            pl.BlockSpec((1, VB), lambda i: (0, i)),
        ],
        out_specs=pl.BlockSpec((VB, B), lambda i: (i, 0)),
        out_shape=jax.ShapeDtypeStruct((VOCAB, B), jnp.float32),
        scratch_shapes=[pltpu.VMEM((EMBED, B), jnp.float32)],
    )(xt, wt, b2d)


def kernel(context_words, emb, W, b):
    cwt = context_words.astype(jnp.int32).T
    xt = _poolt(cwt, emb.T)
    outt = _proj(xt, W.T, b.reshape(1, VOCAB))
    return outt.T


# speedup vs baseline: 1.3077x; 1.0103x over previous
import functools

import jax
import jax.numpy as jnp
from jax import lax
from jax.experimental import pallas as pl
from jax.experimental.pallas import tpu as pltpu
from jax.experimental.pallas import tpu_sc as plsc

VOCAB = 100000
EMBED = 64
B = 1024
L = 50

NC = 2
NS = 16
NW = NC * NS
DPW = EMBED // NW

CB = 256
VB = 4096


def _poolt_body(cwt_hbm, embt_hbm, out_hbm, row_v, cw_v, acc_v, sem):
    wid = lax.axis_index("s") * NC + lax.axis_index("c")
    for dl in range(DPW):
        d = DPW * wid + dl
        pltpu.sync_copy(embt_hbm.at[d], row_v)
        for bc in range(B // CB):
            pltpu.sync_copy(cwt_hbm.at[:, pl.ds(bc * CB, CB)], cw_v)

            def gbody(g, carry, dl=dl, bc=bc):
                acc = jnp.zeros((16,), jnp.float32)
                for i in range(L):
                    idx = cw_v[i, pl.ds(16 * g, 16)]
                    acc = acc + plsc.load_gather(row_v, [idx])
                acc_v[dl, pl.ds(bc * CB + 16 * g, 16)] = acc
                return carry

            lax.fori_loop(0, CB // 16, gbody, 0)
    pltpu.sync_copy(acc_v, out_hbm.at[pl.ds(DPW * wid, DPW)])


@jax.jit
def _poolt(cwt, embt):
    return pl.kernel(
        _poolt_body,
        out_type=jax.ShapeDtypeStruct((EMBED, B), jnp.float32),
        mesh=plsc.VectorSubcoreMesh(core_axis_name="c", subcore_axis_name="s"),
        scratch_types=[
            pltpu.VMEM((VOCAB,), jnp.float32),
            pltpu.VMEM((L, CB), jnp.int32),
            pltpu.VMEM((DPW, B), jnp.float32),
            pltpu.SemaphoreType.DMA,
        ],
        compiler_params=pltpu.CompilerParams(
            use_tc_tiling_on_sc=True, needs_layout_passes=False),
    )(cwt, embt)


def _proj_body(xt_ref, wt_ref, b_ref, outt_ref, xn_ref):
    @pl.when(pl.program_id(0) == 0)
    def _():
        xt = xt_ref[...]
        mu = jnp.mean(xt, axis=1, keepdims=True)
        xc = xt - mu
        var = jnp.mean(xc * xc, axis=1, keepdims=True)
        xn_ref[...] = xc * lax.rsqrt(var + 1e-10)

    acc = lax.dot_general(
        wt_ref[...], xn_ref[...],
        (((0,), (0,)), ((), ())),
        preferred_element_type=jnp.float32,
    )
    bias = lax.dot_general(
        b_ref[...], jnp.ones((1, B), jnp.float32),
        (((0,), (0,)), ((), ())),
        preferred_element_type=jnp.float32,
    )
    outt_ref[...] = acc + bias


@jax.jit
def _proj(xt, wt, b2d):
    grid = (pl.cdiv(VOCAB, VB),)
    return pl.pallas_call(
        _proj_body,
        grid=grid,
        in_specs=[
            pl.BlockSpec((EMBED, B), lambda i: (0, 0)),
            pl.BlockSpec((EMBED, VB), lambda i: (0, i)),
            pl.BlockSpec((1, VB), lambda i: (0, i)),
        ],
        out_specs=pl.BlockSpec((VB, B), lambda i: (i, 0)),
        out_shape=jax.ShapeDtypeStruct((VOCAB, B), jnp.float32),
        scratch_shapes=[pltpu.VMEM((EMBED, B), jnp.float32)],
    )(xt, wt, b2d)


def kernel(context_words, emb, W, b):
    cwt = context_words.astype(jnp.int32).T
    xt = _poolt(cwt, emb.T)
    outt = _proj(xt, W.T, b.reshape(1, VOCAB))
    return outt.T
